# Initial kernel scaffold; baseline (speedup 1.0000x reference)
import jax
import jax.numpy as jnp
from jax import lax
from jax.experimental import pallas as pl

_ITERS = 10
_TOPK = 4


def _kmeans_fusion_kernel(pts_ref, c0_ref, trans_ref, inst_ref,
                          protos_ref, fused_ref):
    n = pts_ref.shape[0]
    k = c0_ref.shape[1]

    px = pts_ref[:, 0:1]
    py = pts_ref[:, 1:2]
    pz = pts_ref[:, 2:3]
    a2 = (px * px + py * py) + pz * pz

    def dist(cT):
        cx = cT[0:1, :]
        cy = cT[1:2, :]
        cz = cT[2:3, :]
        b2 = (cx * cx + cy * cy) + cz * cz
        ab = (px * cx + py * cy) + pz * cz
        return jnp.sqrt(jnp.maximum((a2 + b2) - 2.0 * ab, 0.0))

    def step(_, cT):
        d = dist(cT)
        rmin = jnp.min(d, axis=1, keepdims=True)
        il = lax.broadcasted_iota(jnp.int32, (n, k), 1)
        amin = jnp.min(jnp.where(d == rmin, il, k), axis=1, keepdims=True)
        oh = (il == amin).astype(jnp.float32)
        cnt = jnp.sum(oh, axis=0, keepdims=True)
        sx = jnp.sum(oh * px, axis=0, keepdims=True)
        sy = jnp.sum(oh * py, axis=0, keepdims=True)
        sz = jnp.sum(oh * pz, axis=0, keepdims=True)
        sums = jnp.concatenate([sx, sy, sz], axis=0)
        return jnp.where(cnt > 0, sums / jnp.maximum(cnt, 1.0), cT)

    cT = lax.fori_loop(0, _ITERS, step, c0_ref[:, :])

    d = dist(cT)

    cmin = jnp.min(d, axis=0, keepdims=True)
    isrc = lax.broadcasted_iota(jnp.int32, (n, k), 0)
    nearest = jnp.min(jnp.where(d == cmin, isrc, n), axis=0, keepdims=True)
    oh_n = (isrc == nearest).astype(jnp.float32)
    protos_ref[:, :] = lax.dot_general(
        oh_n, trans_ref[:, :], (((0,), (0,)), ((), ())),
        precision=lax.Precision.HIGHEST,
        preferred_element_type=jnp.float32)

    dt = d[0:k, :]
    il9 = lax.broadcasted_iota(jnp.int32, (k, k), 1)

    def tstep(_, carry):
        w, dcur = carry
        rmin = jnp.min(dcur, axis=1, keepdims=True)
        amin = jnp.min(jnp.where(dcur == rmin, il9, k), axis=1, keepdims=True)
        sel = (il9 == amin)
        return (w + sel.astype(jnp.float32),
                jnp.where(sel, jnp.float32(jnp.inf), dcur))

    w0 = jnp.zeros((k, k), jnp.float32)
    w, _ = lax.fori_loop(0, _TOPK, tstep, (w0, dt))
    fused_ref[:, :] = lax.dot_general(
        w, inst_ref[:, :], (((1,), (0,)), ((), ())),
        precision=lax.Precision.HIGHEST,
        preferred_element_type=jnp.float32)


def kernel(ego_anchor, trans_anchor, ego_feature, instance_feature):
    N, A, D = trans_anchor.shape
    E = instance_feature.shape[-1]
    trans_flat = trans_anchor.reshape(N * A, D)
    pts = trans_flat[:, :3]
    c0T = jnp.transpose(pts[:: (N * A) // A])
    inst0 = instance_feature.reshape(N * A, E)[:A]

    protos, fused = pl.pallas_call(
        _kmeans_fusion_kernel,
        out_shape=(jax.ShapeDtypeStruct((A, D), jnp.float32),
                   jax.ShapeDtypeStruct((A, E), jnp.float32)),
    )(pts, c0T, trans_flat, inst0)
    return protos, fused


# speedup vs baseline: 7.1180x; 7.1180x over previous
import jax
import jax.numpy as jnp
from jax import lax
from jax.experimental import pallas as pl

_ITERS = 10
_TOPK = 4


def _kmeans_fusion_kernel(ptsT_ref, p1_ref, c0_ref, trans_ref, inst_ref,
                          protos_ref, fused_ref):
    k = c0_ref.shape[0]
    n = ptsT_ref.shape[1]

    ptsT = ptsT_ref[:, :]
    a2 = jnp.sum(ptsT * ptsT, axis=0, keepdims=True)
    p1 = p1_ref[:, :]

    def dist(c):
        b2 = jnp.sum(c * c, axis=1, keepdims=True)
        ab = lax.dot_general(c, ptsT, (((1,), (0,)), ((), ())),
                             preferred_element_type=jnp.float32)
        return jnp.sqrt(jnp.maximum(a2 + b2 - 2.0 * ab, 0.0))

    def step(_, c):
        d = dist(c)
        cmin = jnp.min(d, axis=0, keepdims=True)
        isub = lax.broadcasted_iota(jnp.int32, (k, n), 0)
        amin = jnp.min(jnp.where(d == cmin, isub, k), axis=0, keepdims=True)
        oh = (isub == amin).astype(jnp.float32)
        cc = lax.dot_general(oh, p1, (((1,), (0,)), ((), ())),
                             precision=lax.Precision.HIGHEST,
                             preferred_element_type=jnp.float32)
        cnt = cc[:, 3:4]
        return jnp.where(cnt > 0, cc[:, 0:3] / jnp.maximum(cnt, 1.0), c)

    c = lax.fori_loop(0, _ITERS, step, c0_ref[:, :])

    d = dist(c)

    rmin = jnp.min(d, axis=1, keepdims=True)
    ilane = lax.broadcasted_iota(jnp.int32, (k, n), 1)
    nearest = jnp.min(jnp.where(d == rmin, ilane, n), axis=1, keepdims=True)
    oh_n = (ilane == nearest).astype(jnp.float32)
    protos_ref[:, :] = lax.dot_general(
        oh_n, trans_ref[:, :], (((1,), (0,)), ((), ())),
        precision=lax.Precision.HIGHEST,
        preferred_element_type=jnp.float32)

    dt = d[:, 0:k]
    is9 = lax.broadcasted_iota(jnp.int32, (k, k), 0)

    def tstep(_, carry):
        wt, dcur = carry
        cmin = jnp.min(dcur, axis=0, keepdims=True)
        amin = jnp.min(jnp.where(dcur == cmin, is9, k), axis=0, keepdims=True)
        sel = (is9 == amin)
        return (wt + sel.astype(jnp.float32),
                jnp.where(sel, jnp.float32(jnp.inf), dcur))

    wt0 = jnp.zeros((k, k), jnp.float32)
    wt, _ = lax.fori_loop(0, _TOPK, tstep, (wt0, dt))
    fused_ref[:, :] = lax.dot_general(
        wt, inst_ref[:, :], (((0,), (0,)), ((), ())),
        precision=lax.Precision.HIGHEST,
        preferred_element_type=jnp.float32)


def kernel(ego_anchor, trans_anchor, ego_feature, instance_feature):
    N, A, D = trans_anchor.shape
    E = instance_feature.shape[-1]
    trans_flat = trans_anchor.reshape(N * A, D)
    pts = trans_flat[:, :3]
    ptsT = jnp.transpose(pts)
    p1 = jnp.concatenate([pts, jnp.ones((N * A, 1), jnp.float32)], axis=1)
    c0 = pts[:: (N * A) // A]
    inst0 = instance_feature.reshape(N * A, E)[:A]

    protos, fused = pl.pallas_call(
        _kmeans_fusion_kernel,
        out_shape=(jax.ShapeDtypeStruct((A, D), jnp.float32),
                   jax.ShapeDtypeStruct((A, E), jnp.float32)),
    )(ptsT, p1, c0, trans_flat, inst0)
    return protos, fused


# speedup vs baseline: 10.3470x; 1.4536x over previous
import jax
import jax.numpy as jnp
from jax import lax
from jax.experimental import pallas as pl

_ITERS = 10
_TOPK = 4


def _kmeans_fusion_kernel(p1_ref, c0_ref, trans_ref, inst_ref,
                          protos_ref, fused_ref):
    n = p1_ref.shape[0]
    k = c0_ref.shape[1]

    p1 = p1_ref[:, :]
    pts = p1[:, 0:3]
    a2 = jnp.sum(pts * pts, axis=1, keepdims=True)

    def dist(cT):
        b2 = jnp.sum(cT * cT, axis=0, keepdims=True)
        ab = lax.dot_general(pts, cT, (((1,), (0,)), ((), ())),
                             preferred_element_type=jnp.float32)
        return jnp.sqrt(jnp.maximum(a2 + b2 - 2.0 * ab, 0.0))

    def step(_, cT):
        d = dist(cT)
        rmin = jnp.min(d, axis=1, keepdims=True)
        il = lax.broadcasted_iota(jnp.int32, (n, k), 1)
        amin = jnp.min(jnp.where(d == rmin, il, k), axis=1, keepdims=True)
        oh = (il == amin).astype(jnp.float32)
        cnt = jnp.sum(oh, axis=0, keepdims=True)
        sx = jnp.sum(oh * p1[:, 0:1], axis=0, keepdims=True)
        sy = jnp.sum(oh * p1[:, 1:2], axis=0, keepdims=True)
        sz = jnp.sum(oh * p1[:, 2:3], axis=0, keepdims=True)
        sums = jnp.concatenate([sx, sy, sz], axis=0)
        return jnp.where(cnt > 0, sums / jnp.maximum(cnt, 1.0), cT)

    cT = lax.fori_loop(0, _ITERS, step, c0_ref[:, :])

    d = dist(cT)

    cmin = jnp.min(d, axis=0, keepdims=True)
    isrc = lax.broadcasted_iota(jnp.int32, (n, k), 0)
    nearest = jnp.min(jnp.where(d == cmin, isrc, n), axis=0, keepdims=True)
    oh_n = (isrc == nearest).astype(jnp.bfloat16)
    protos_ref[:, :] = lax.dot_general(
        oh_n, trans_ref[:, :], (((0,), (0,)), ((), ())),
        preferred_element_type=jnp.float32)

    dt = d[0:k, :]
    il9 = lax.broadcasted_iota(jnp.int32, (k, k), 1)

    def tstep(_, carry):
        w, dcur = carry
        rmin = jnp.min(dcur, axis=1, keepdims=True)
        amin = jnp.min(jnp.where(dcur == rmin, il9, k), axis=1, keepdims=True)
        sel = (il9 == amin)
        return (w + sel.astype(jnp.float32),
                jnp.where(sel, jnp.float32(jnp.inf), dcur))

    w0 = jnp.zeros((k, k), jnp.float32)
    w, _ = lax.fori_loop(0, _TOPK, tstep, (w0, dt))
    fused_ref[:, :] = lax.dot_general(
        w.astype(jnp.bfloat16), inst_ref[:, :], (((1,), (0,)), ((), ())),
        preferred_element_type=jnp.float32)


def kernel(ego_anchor, trans_anchor, ego_feature, instance_feature):
    N, A, D = trans_anchor.shape
    E = instance_feature.shape[-1]
    trans_flat = trans_anchor.reshape(N * A, D)
    pts = trans_flat[:, :3]
    p1 = jnp.concatenate([pts, jnp.ones((N * A, 1), jnp.float32)], axis=1)
    c0T = jnp.transpose(pts[:: (N * A) // A])
    inst0 = instance_feature.reshape(N * A, E)[:A]

    protos, fused = pl.pallas_call(
        _kmeans_fusion_kernel,
        out_shape=(jax.ShapeDtypeStruct((A, D), jnp.float32),
                   jax.ShapeDtypeStruct((A, E), jnp.float32)),
    )(p1, c0T, trans_flat, inst0)
    return protos, fused
